# Initial kernel scaffold; baseline (speedup 1.0000x reference)
import jax
import jax.numpy as jnp
from jax.experimental import pallas as pl


def kernel(center_words, context_words, neg_samples, in_embed, out_embed):
    raise NotImplementedError("write your pallas kernel here")


# speedup vs baseline: 3.2335x; 3.2335x over previous
import functools

import jax
import jax.numpy as jnp
from jax import lax
from jax.experimental import pallas as pl
from jax.experimental.pallas import tpu as pltpu
from jax.experimental.pallas import tpu_sc as plsc

V = 100000
D = 400
B = 16384
K = 20

NC = 2
NS = 16
NW = NC * NS
BPW = B // NW
C = 8
NCHUNK = BPW // C
CK = C * K
H = CK // 2
DJ = D // 16

_mesh = plsc.VectorSubcoreMesh(core_axis_name="c", subcore_axis_name="s")

_GDN = lax.GatherDimensionNumbers(
    offset_dims=(), collapsed_slice_dims=(0,), start_index_map=(0,))


def _lane_shuffle(a, idx):
    return lax.gather(a, idx[:, None], _GDN, slice_sizes=(1,),
                      mode=lax.GatherScatterMode.PROMISE_IN_BOUNDS)


def _hsum(a):
    iota = lax.iota(jnp.int32, 16)
    for sh in (8, 4, 2, 1):
        a = a + _lane_shuffle(a, jnp.bitwise_xor(iota, sh))
    return a


@functools.partial(
    pl.kernel,
    mesh=_mesh,
    compiler_params=pltpu.CompilerParams(
        needs_layout_passes=False, use_tc_tiling_on_sc=False),
    out_type=[
        jax.ShapeDtypeStruct((B,), jnp.float32),
        jax.ShapeDtypeStruct((B * K,), jnp.float32),
    ],
    scratch_types=[
        pltpu.VMEM((C,), jnp.int32),
        pltpu.VMEM((C,), jnp.int32),
        pltpu.VMEM((H,), jnp.int32),
        pltpu.VMEM((H,), jnp.int32),
        pltpu.VMEM((C, D), jnp.float32),
        pltpu.VMEM((C, D), jnp.float32),
        pltpu.VMEM((CK, D), jnp.float32),
        pltpu.VMEM((16,), jnp.float32),
        pltpu.VMEM((CK,), jnp.float32),
        pltpu.SemaphoreType.DMA,
    ],
)
def _sc_scores(cw_hbm, xw_hbm, nw_hbm, in_hbm, out_hbm,
               pos_hbm, negs_hbm,
               cidx, xidx, nidx0, nidx1, cen, ctx, neg, posb, negb, sem):
    wid = lax.axis_index("s") * NC + lax.axis_index("c")
    base = wid * BPW

    def chunk(i, carry):
        row0 = base + i * C
        pltpu.sync_copy(cw_hbm.at[pl.ds(row0, C)], cidx)
        pltpu.sync_copy(xw_hbm.at[pl.ds(row0, C)], xidx)
        pltpu.sync_copy(nw_hbm.at[pl.ds(row0 * K, H)], nidx0)
        pltpu.sync_copy(nw_hbm.at[pl.ds(row0 * K + H, H)], nidx1)
        c1 = pltpu.async_copy(in_hbm.at[cidx], cen, sem)
        c2 = pltpu.async_copy(out_hbm.at[xidx], ctx, sem)
        c3 = pltpu.async_copy(out_hbm.at[nidx0], neg.at[pl.ds(0, H)], sem)
        c4 = pltpu.async_copy(out_hbm.at[nidx1], neg.at[pl.ds(H, H)], sem)
        c1.wait()
        c2.wait()
        c3.wait()
        c4.wait()

        iota = lax.iota(jnp.int32, 16)

        def row_body(r, posvec):
            cvec = [cen[r, pl.ds(j * 16, 16)] for j in range(DJ)]
            acc = cvec[0] * ctx[r, pl.ds(0, 16)]
            for j in range(1, DJ):
                acc = acc + cvec[j] * ctx[r, pl.ds(j * 16, 16)]
            posvec = jnp.where(iota == r, _hsum(acc), posvec)

            vecA = jnp.zeros((16,), jnp.float32)
            vecB = jnp.zeros((16,), jnp.float32)
            for k in range(K):
                q = r * K + k
                a = cvec[0] * neg[q, pl.ds(0, 16)]
                for j in range(1, DJ):
                    a = a + cvec[j] * neg[q, pl.ds(j * 16, 16)]
                s = _hsum(a)
                if k < 16:
                    vecA = jnp.where(iota == k, s, vecA)
                else:
                    vecB = jnp.where(iota == (k - 16), s, vecB)
            plsc.store_scatter(negb, [r * K + iota], vecA)
            plsc.store_scatter(negb, [r * K + 16 + iota], vecB,
                               mask=iota < (K - 16))
            return posvec

        posvec = lax.fori_loop(0, C, row_body, jnp.zeros((16,), jnp.float32))
        posb[...] = posvec

        pltpu.sync_copy(posb.at[pl.ds(0, C)], pos_hbm.at[pl.ds(row0, C)])
        pltpu.sync_copy(negb, negs_hbm.at[pl.ds(row0 * K, CK)])
        return carry

    lax.fori_loop(0, NCHUNK, chunk, 0)


def _loss_body(pos_ref, neg_ref, out_ref):
    p = pos_ref[...]
    n = neg_ref[...]

    def softplus(z):
        return jnp.maximum(z, 0.0) + jnp.log(1.0 + jnp.exp(-jnp.abs(z)))

    total = jnp.sum(softplus(-p)) + jnp.sum(softplus(n))
    out_ref[0, 0] = total / B


def kernel(center_words, context_words, neg_samples, in_embed, out_embed):
    cw = center_words.astype(jnp.int32)
    xw = context_words.astype(jnp.int32)
    nw = neg_samples.astype(jnp.int32).reshape(B * K)
    pos, negs = _sc_scores(cw, xw, nw, in_embed, out_embed)
    loss = pl.pallas_call(
        _loss_body,
        out_shape=jax.ShapeDtypeStruct((1, 1), jnp.float32),
        out_specs=pl.BlockSpec(memory_space=pltpu.SMEM),
    )(pos.reshape(128, 128), negs.reshape(2560, 128))
    return loss[0, 0]


# speedup vs baseline: 3.6454x; 1.1274x over previous
import functools

import jax
import jax.numpy as jnp
from jax import lax
from jax.experimental import pallas as pl
from jax.experimental.pallas import tpu as pltpu
from jax.experimental.pallas import tpu_sc as plsc

V = 100000
D = 400
B = 16384
K = 20

NC = 2
NS = 16
NW = NC * NS
BPW = B // NW
HC = 8
NHC = BPW // HC
HK = HC * K
HKH = HK // 2
DJ = D // 16

_mesh = plsc.VectorSubcoreMesh(core_axis_name="c", subcore_axis_name="s")

_GDN = lax.GatherDimensionNumbers(
    offset_dims=(), collapsed_slice_dims=(0,), start_index_map=(0,))


def _lane_shuffle(a, idx):
    return lax.gather(a, idx[:, None], _GDN, slice_sizes=(1,),
                      mode=lax.GatherScatterMode.PROMISE_IN_BOUNDS)


def _hsum(a):
    iota = lax.iota(jnp.int32, 16)
    for sh in (8, 4, 2, 1):
        a = a + _lane_shuffle(a, jnp.bitwise_xor(iota, sh))
    return a


@functools.partial(
    pl.kernel,
    mesh=_mesh,
    compiler_params=pltpu.CompilerParams(
        needs_layout_passes=False, use_tc_tiling_on_sc=False),
    out_type=[
        jax.ShapeDtypeStruct((B,), jnp.float32),
        jax.ShapeDtypeStruct((B * K,), jnp.float32),
    ],
    scratch_types=[
        pltpu.VMEM((BPW,), jnp.int32),
        pltpu.VMEM((BPW,), jnp.int32),
        pltpu.VMEM((BPW * K,), jnp.int32),
        pltpu.VMEM((HC, D), jnp.float32),
        pltpu.VMEM((HC, D), jnp.float32),
        pltpu.VMEM((HK, D), jnp.float32),
        pltpu.VMEM((BPW,), jnp.float32),
        pltpu.VMEM((BPW * K,), jnp.float32),
        pltpu.SemaphoreType.DMA,
    ],
)
def _sc_scores(cw_hbm, xw_hbm, nw_hbm, in_hbm, out_hbm,
               pos_hbm, negs_hbm,
               cidx, xidx, nidx, cen, ctx, neg, posb, negb, sem):
    wid = lax.axis_index("s") * NC + lax.axis_index("c")
    base = wid * BPW
    iota = lax.iota(jnp.int32, 16)
    zeros = jnp.zeros((16,), jnp.int32)

    pltpu.sync_copy(cw_hbm.at[pl.ds(base, BPW)], cidx)
    pltpu.sync_copy(xw_hbm.at[pl.ds(base, BPW)], xidx)
    pltpu.sync_copy(nw_hbm.at[pl.ds(base * K, BPW * K)], nidx)

    def step(s, carry):
        c0 = pltpu.async_copy(in_hbm.at[cidx.at[pl.ds(s * HC, HC)]],
                              cen, sem)
        c1 = pltpu.async_copy(out_hbm.at[xidx.at[pl.ds(s * HC, HC)]],
                              ctx, sem)
        c2 = pltpu.async_copy(out_hbm.at[nidx.at[pl.ds(s * HK, HKH)]],
                              neg.at[pl.ds(0, HKH)], sem)
        c3 = pltpu.async_copy(out_hbm.at[nidx.at[pl.ds(s * HK + HKH, HKH)]],
                              neg.at[pl.ds(HKH, HKH)], sem)
        c0.wait()
        c1.wait()
        c2.wait()
        c3.wait()

        def row(r, rcarry):
            g = s * HC + r
            cvec = [cen[r, pl.ds(j * 16, 16)] for j in range(DJ)]
            acc = cvec[0] * ctx[r, pl.ds(0, 16)]
            for j in range(1, DJ):
                acc = acc + cvec[j] * ctx[r, pl.ds(j * 16, 16)]
            plsc.store_scatter(posb, [zeros + g], _hsum(acc),
                               mask=iota == 0)

            vecA = jnp.zeros((16,), jnp.float32)
            vecB = jnp.zeros((16,), jnp.float32)
            for k in range(K):
                q = r * K + k
                a = cvec[0] * neg[q, pl.ds(0, 16)]
                for j in range(1, DJ):
                    a = a + cvec[j] * neg[q, pl.ds(j * 16, 16)]
                sv = _hsum(a)
                if k < 16:
                    vecA = jnp.where(iota == k, sv, vecA)
                else:
                    vecB = jnp.where(iota == (k - 16), sv, vecB)
            plsc.store_scatter(negb, [g * K + iota], vecA)
            plsc.store_scatter(negb, [g * K + 16 + iota], vecB,
                               mask=iota < (K - 16))
            return rcarry

        lax.fori_loop(0, HC, row, 0)
        return carry

    lax.fori_loop(0, NHC, step, 0)

    pltpu.sync_copy(posb, pos_hbm.at[pl.ds(base, BPW)])
    pltpu.sync_copy(negb, negs_hbm.at[pl.ds(base * K, BPW * K)])


def _loss_body(pos_ref, neg_ref, out_ref):
    p = pos_ref[...]
    n = neg_ref[...]

    def softplus(z):
        return jnp.maximum(z, 0.0) + jnp.log(1.0 + jnp.exp(-jnp.abs(z)))

    total = jnp.sum(softplus(-p)) + jnp.sum(softplus(n))
    out_ref[0, 0] = total / B


def kernel(center_words, context_words, neg_samples, in_embed, out_embed):
    cw = center_words.astype(jnp.int32)
    xw = context_words.astype(jnp.int32)
    nw = neg_samples.astype(jnp.int32).reshape(B * K)
    pos, negs = _sc_scores(cw, xw, nw, in_embed, out_embed)
    loss = pl.pallas_call(
        _loss_body,
        out_shape=jax.ShapeDtypeStruct((1, 1), jnp.float32),
        out_specs=pl.BlockSpec(memory_space=pltpu.SMEM),
    )(pos.reshape(128, 128), negs.reshape(2560, 128))
    return loss[0, 0]
